# baseline (speedup 1.0000x reference)
import functools

import jax
import jax.numpy as jnp
from jax.experimental import pallas as pl
from jax.experimental.pallas import tpu as pltpu

_K = 8
_BLK = 256


def _router_block(r_ref, hn_ref, cf_ref, sf_ref, w1_ref, whn_ref, wcf_ref,
                  wsf_ref, b1_ref, w2_ref, b2_ref, w_out_ref, l_out_ref):
    bf = lambda x: x.astype(jnp.bfloat16)
    bfc = lambda x: x.astype(jnp.bfloat16).astype(jnp.float32)
    r = bf(r_ref[...])
    h = jnp.dot(r, bf(w1_ref[...]), preferred_element_type=jnp.float32)
    h = h + bfc(hn_ref[...]) * bfc(whn_ref[...])
    h = h + bfc(cf_ref[...]) * bfc(wcf_ref[...])
    h = h + (bfc(sf_ref[...]) * bfc(wsf_ref[...]) + b1_ref[...])
    h = h * jax.nn.sigmoid(h)
    logits = jnp.dot(bf(h), bf(w2_ref[...]), preferred_element_type=jnp.float32)
    logits = logits + b2_ref[...]
    l_out_ref[...] = logits

    cur = logits
    top0 = None
    thresh = None
    for _ in range(_K):
        thresh = jnp.max(cur, axis=-1, keepdims=True)
        cur = jnp.where(cur == thresh, jnp.float32(-jnp.inf), cur)
        if top0 is None:
            top0 = thresh
    exps = jnp.where(logits >= thresh, jnp.exp(logits - top0),
                     jnp.float32(0.0))
    denom = jnp.sum(exps, axis=-1, keepdims=True)
    w_out_ref[...] = exps / denom


@jax.jit
def kernel(r_pooled, step_frac, hidden_norm, confidence, W1, b1, W2, b2):
    bn, d = r_pooled.shape
    h = W1.shape[1]
    e = W2.shape[1]
    w1_main = W1[:d]
    wsf = W1[d][None, :]
    whn = W1[d + 1][None, :]
    wcf = W1[d + 2][None, :]
    b1r = b1[None, :]
    sfv = jnp.asarray(step_frac, jnp.float32).reshape(1, 1)
    hn = hidden_norm[:, None]
    cf = confidence[:, None]
    b2r = b2[None, :]

    grid = (bn // _BLK,)
    full = lambda *shape: pl.BlockSpec(shape, lambda i: (0,) * len(shape))
    rows = lambda w: pl.BlockSpec((_BLK, w), lambda i: (i, 0))
    out_shapes = (
        jax.ShapeDtypeStruct((bn, e), jnp.float32),
        jax.ShapeDtypeStruct((bn, e), jnp.float32),
    )
    weights, logits = pl.pallas_call(
        _router_block,
        grid=grid,
        in_specs=[
            rows(d),
            rows(1),
            rows(1),
            full(1, 1),
            full(d, h),
            full(1, h),
            full(1, h),
            full(1, h),
            full(1, h),
            full(h, e),
            full(1, e),
        ],
        out_specs=(rows(e), rows(e)),
        out_shape=out_shapes,
    )(r_pooled, hn, cf, sfv, w1_main, whn, wcf, wsf, b1r, W2, b2r)
    return weights, logits


# speedup vs baseline: 1.4043x; 1.4043x over previous
import functools

import jax
import jax.numpy as jnp
from jax.experimental import pallas as pl
from jax.experimental.pallas import tpu as pltpu

_K = 8
_BLK = 512


def _router_block(r_ref, hn_ref, cf_ref, sf_ref, w1_ref, whn_ref, wcf_ref,
                  wsf_ref, b1_ref, w2_ref, b2_ref, w_out_ref, l_out_ref):
    bf = lambda x: x.astype(jnp.bfloat16)
    bfc = lambda x: x.astype(jnp.bfloat16).astype(jnp.float32)
    r = bf(r_ref[...])
    h = jnp.dot(r, bf(w1_ref[...]), preferred_element_type=jnp.float32)
    h = h + bfc(hn_ref[...]) * bfc(whn_ref[...])
    h = h + bfc(cf_ref[...]) * bfc(wcf_ref[...])
    h = h + (bfc(sf_ref[...]) * bfc(wsf_ref[...]) + b1_ref[...])
    h = h * jax.nn.sigmoid(h)
    logits = jnp.dot(bf(h), bf(w2_ref[...]), preferred_element_type=jnp.float32)
    logits = logits + b2_ref[...]
    l_out_ref[...] = logits

    cur = logits
    top0 = None
    thresh = None
    for _ in range(_K):
        thresh = jnp.max(cur, axis=-1, keepdims=True)
        cur = jnp.where(cur == thresh, jnp.float32(-jnp.inf), cur)
        if top0 is None:
            top0 = thresh
    exps = jnp.where(logits >= thresh, jnp.exp(logits - top0),
                     jnp.float32(0.0))
    denom = jnp.sum(exps, axis=-1, keepdims=True)
    w_out_ref[...] = exps / denom


@jax.jit
def kernel(r_pooled, step_frac, hidden_norm, confidence, W1, b1, W2, b2):
    bn, d = r_pooled.shape
    h = W1.shape[1]
    e = W2.shape[1]
    w1_main = W1[:d]
    wsf = W1[d][None, :]
    whn = W1[d + 1][None, :]
    wcf = W1[d + 2][None, :]
    b1r = b1[None, :]
    sfv = jnp.asarray(step_frac, jnp.float32).reshape(1, 1)
    hn = hidden_norm[:, None]
    cf = confidence[:, None]
    b2r = b2[None, :]

    grid = (bn // _BLK,)
    full = lambda *shape: pl.BlockSpec(shape, lambda i: (0,) * len(shape))
    rows = lambda w: pl.BlockSpec((_BLK, w), lambda i: (i, 0))
    out_shapes = (
        jax.ShapeDtypeStruct((bn, e), jnp.float32),
        jax.ShapeDtypeStruct((bn, e), jnp.float32),
    )
    weights, logits = pl.pallas_call(
        _router_block,
        grid=grid,
        in_specs=[
            rows(d),
            rows(1),
            rows(1),
            full(1, 1),
            full(d, h),
            full(1, h),
            full(1, h),
            full(1, h),
            full(1, h),
            full(h, e),
            full(1, e),
        ],
        out_specs=(rows(e), rows(e)),
        out_shape=out_shapes,
    )(r_pooled, hn, cf, sfv, w1_main, whn, wcf, wsf, b1r, W2, b2r)
    return weights, logits


# speedup vs baseline: 1.8408x; 1.3109x over previous
import functools

import jax
import jax.numpy as jnp
from jax.experimental import pallas as pl
from jax.experimental.pallas import tpu as pltpu

_K = 8
_BLK = 2048


def _router_block(r_ref, hn_ref, cf_ref, sf_ref, w1_ref, whn_ref, wcf_ref,
                  wsf_ref, b1_ref, w2_ref, b2_ref, w_out_ref, l_out_ref):
    bf = lambda x: x.astype(jnp.bfloat16)
    bfc = lambda x: x.astype(jnp.bfloat16).astype(jnp.float32)
    r = bf(r_ref[...])
    h = jnp.dot(r, bf(w1_ref[...]), preferred_element_type=jnp.float32)
    h = h + bfc(hn_ref[...]) * bfc(whn_ref[...])
    h = h + bfc(cf_ref[...]) * bfc(wcf_ref[...])
    h = h + (bfc(sf_ref[...]) * bfc(wsf_ref[...]) + b1_ref[...])
    h = h * jax.nn.sigmoid(h)
    logits = jnp.dot(bf(h), bf(w2_ref[...]), preferred_element_type=jnp.float32)
    logits = logits + b2_ref[...]
    l_out_ref[...] = logits

    cur = logits
    top0 = None
    thresh = None
    for _ in range(_K):
        thresh = jnp.max(cur, axis=-1, keepdims=True)
        cur = jnp.where(cur == thresh, jnp.float32(-jnp.inf), cur)
        if top0 is None:
            top0 = thresh
    exps = jnp.where(logits >= thresh, jnp.exp(logits - top0),
                     jnp.float32(0.0))
    denom = jnp.sum(exps, axis=-1, keepdims=True)
    w_out_ref[...] = exps / denom


@jax.jit
def kernel(r_pooled, step_frac, hidden_norm, confidence, W1, b1, W2, b2):
    bn, d = r_pooled.shape
    h = W1.shape[1]
    e = W2.shape[1]
    w1_main = W1[:d]
    wsf = W1[d][None, :]
    whn = W1[d + 1][None, :]
    wcf = W1[d + 2][None, :]
    b1r = b1[None, :]
    sfv = jnp.asarray(step_frac, jnp.float32).reshape(1, 1)
    hn = hidden_norm[:, None]
    cf = confidence[:, None]
    b2r = b2[None, :]

    grid = (bn // _BLK,)
    full = lambda *shape: pl.BlockSpec(shape, lambda i: (0,) * len(shape))
    rows = lambda w: pl.BlockSpec((_BLK, w), lambda i: (i, 0))
    out_shapes = (
        jax.ShapeDtypeStruct((bn, e), jnp.float32),
        jax.ShapeDtypeStruct((bn, e), jnp.float32),
    )
    weights, logits = pl.pallas_call(
        _router_block,
        grid=grid,
        in_specs=[
            rows(d),
            rows(1),
            rows(1),
            full(1, 1),
            full(d, h),
            full(1, h),
            full(1, h),
            full(1, h),
            full(1, h),
            full(h, e),
            full(1, e),
        ],
        out_specs=(rows(e), rows(e)),
        out_shape=out_shapes,
    )(r_pooled, hn, cf, sfv, w1_main, whn, wcf, wsf, b1r, W2, b2r)
    return weights, logits


# speedup vs baseline: 1.8857x; 1.0244x over previous
import functools

import jax
import jax.numpy as jnp
from jax.experimental import pallas as pl
from jax.experimental.pallas import tpu as pltpu

_K = 8
_BLK = 4096


def _router_block(r_ref, hn_ref, cf_ref, sf_ref, w1_ref, whn_ref, wcf_ref,
                  wsf_ref, b1_ref, w2_ref, b2_ref, w_out_ref, l_out_ref):
    bf = lambda x: x.astype(jnp.bfloat16)
    bfc = lambda x: x.astype(jnp.bfloat16).astype(jnp.float32)
    r = bf(r_ref[...])
    h = jnp.dot(r, bf(w1_ref[...]), preferred_element_type=jnp.float32)
    h = h + bfc(hn_ref[...]) * bfc(whn_ref[...])
    h = h + bfc(cf_ref[...]) * bfc(wcf_ref[...])
    h = h + (bfc(sf_ref[...]) * bfc(wsf_ref[...]) + b1_ref[...])
    h = h * jax.nn.sigmoid(h)
    logits = jnp.dot(bf(h), bf(w2_ref[...]), preferred_element_type=jnp.float32)
    logits = logits + b2_ref[...]
    l_out_ref[...] = logits

    cur = logits
    top0 = None
    thresh = None
    for _ in range(_K):
        thresh = jnp.max(cur, axis=-1, keepdims=True)
        cur = jnp.where(cur == thresh, jnp.float32(-jnp.inf), cur)
        if top0 is None:
            top0 = thresh
    exps = jnp.where(logits >= thresh, jnp.exp(logits - top0),
                     jnp.float32(0.0))
    denom = jnp.sum(exps, axis=-1, keepdims=True)
    w_out_ref[...] = exps / denom


@jax.jit
def kernel(r_pooled, step_frac, hidden_norm, confidence, W1, b1, W2, b2):
    bn, d = r_pooled.shape
    h = W1.shape[1]
    e = W2.shape[1]
    w1_main = W1[:d]
    wsf = W1[d][None, :]
    whn = W1[d + 1][None, :]
    wcf = W1[d + 2][None, :]
    b1r = b1[None, :]
    sfv = jnp.asarray(step_frac, jnp.float32).reshape(1, 1)
    hn = hidden_norm[:, None]
    cf = confidence[:, None]
    b2r = b2[None, :]

    grid = (bn // _BLK,)
    full = lambda *shape: pl.BlockSpec(shape, lambda i: (0,) * len(shape))
    rows = lambda w: pl.BlockSpec((_BLK, w), lambda i: (i, 0))
    out_shapes = (
        jax.ShapeDtypeStruct((bn, e), jnp.float32),
        jax.ShapeDtypeStruct((bn, e), jnp.float32),
    )
    weights, logits = pl.pallas_call(
        _router_block,
        grid=grid,
        in_specs=[
            rows(d),
            rows(1),
            rows(1),
            full(1, 1),
            full(d, h),
            full(1, h),
            full(1, h),
            full(1, h),
            full(1, h),
            full(h, e),
            full(1, e),
        ],
        out_specs=(rows(e), rows(e)),
        out_shape=out_shapes,
    )(r_pooled, hn, cf, sfv, w1_main, whn, wcf, wsf, b1r, W2, b2r)
    return weights, logits


# speedup vs baseline: 1.8909x; 1.0028x over previous
import functools

import jax
import jax.numpy as jnp
from jax.experimental import pallas as pl
from jax.experimental.pallas import tpu as pltpu

_K = 8
_BLK = 4096


def _router_block(r_ref, hn_ref, cf_ref, sf_ref, w1_ref, whn_ref, wcf_ref,
                  wsf_ref, b1_ref, w2_ref, b2_ref, w_out_ref, l_out_ref):
    bf = lambda x: x.astype(jnp.bfloat16)
    bfc = lambda x: x.astype(jnp.bfloat16).astype(jnp.float32)
    r = bf(r_ref[...])
    h = jnp.dot(r, bf(w1_ref[...]), preferred_element_type=jnp.float32)
    h = h + bfc(hn_ref[...]) * bfc(whn_ref[...])
    h = h + bfc(cf_ref[...]) * bfc(wcf_ref[...])
    h = h + (bfc(sf_ref[...]) * bfc(wsf_ref[...]) + b1_ref[...])
    h = h * jax.nn.sigmoid(h)
    logits = jnp.dot(bf(h), bf(w2_ref[...]), preferred_element_type=jnp.float32)
    logits = logits + b2_ref[...]
    l_out_ref[...] = logits

    neg = jnp.float32(-jnp.inf)
    top0 = jnp.max(logits, axis=-1, keepdims=True)
    thresh = top0
    for _ in range(_K - 1):
        thresh = jnp.max(jnp.where(logits < thresh, logits, neg),
                         axis=-1, keepdims=True)
    exps = jnp.where(logits >= thresh, jnp.exp(logits - top0),
                     jnp.float32(0.0))
    denom = jnp.sum(exps, axis=-1, keepdims=True)
    w_out_ref[...] = exps / denom


@jax.jit
def kernel(r_pooled, step_frac, hidden_norm, confidence, W1, b1, W2, b2):
    bn, d = r_pooled.shape
    h = W1.shape[1]
    e = W2.shape[1]
    w1_main = W1[:d]
    wsf = W1[d][None, :]
    whn = W1[d + 1][None, :]
    wcf = W1[d + 2][None, :]
    b1r = b1[None, :]
    sfv = jnp.asarray(step_frac, jnp.float32).reshape(1, 1)
    hn = hidden_norm[:, None]
    cf = confidence[:, None]
    b2r = b2[None, :]

    grid = (bn // _BLK,)
    full = lambda *shape: pl.BlockSpec(shape, lambda i: (0,) * len(shape))
    rows = lambda w: pl.BlockSpec((_BLK, w), lambda i: (i, 0))
    out_shapes = (
        jax.ShapeDtypeStruct((bn, e), jnp.float32),
        jax.ShapeDtypeStruct((bn, e), jnp.float32),
    )
    weights, logits = pl.pallas_call(
        _router_block,
        grid=grid,
        in_specs=[
            rows(d),
            rows(1),
            rows(1),
            full(1, 1),
            full(d, h),
            full(1, h),
            full(1, h),
            full(1, h),
            full(1, h),
            full(h, e),
            full(1, e),
        ],
        out_specs=(rows(e), rows(e)),
        out_shape=out_shapes,
    )(r_pooled, hn, cf, sfv, w1_main, whn, wcf, wsf, b1r, W2, b2r)
    return weights, logits


# speedup vs baseline: 2.1301x; 1.1265x over previous
import functools

import jax
import jax.numpy as jnp
from jax.experimental import pallas as pl
from jax.experimental.pallas import tpu as pltpu

_K = 8
_BLK = 4096


def _router_block(r_ref, hc_ref, sf_ref, w1_ref, whn_ref, wcf_ref,
                  wsf_ref, b1_ref, w2_ref, b2_ref, w_out_ref, l_out_ref):
    bf = lambda x: x.astype(jnp.bfloat16)
    bfc = lambda x: x.astype(jnp.bfloat16).astype(jnp.float32)
    r = bf(r_ref[...])
    h = jnp.dot(r, bf(w1_ref[...]), preferred_element_type=jnp.float32)
    hc = hc_ref[...].astype(jnp.float32)
    h = h + hc[:, 0:1] * bfc(whn_ref[...])
    h = h + hc[:, 1:2] * bfc(wcf_ref[...])
    h = h + (bfc(sf_ref[...]) * bfc(wsf_ref[...]) + b1_ref[...])
    h = h * jax.nn.sigmoid(h)
    logits = jnp.dot(bf(h), bf(w2_ref[...]), preferred_element_type=jnp.float32)
    logits = logits + b2_ref[...]
    l_out_ref[...] = logits

    neg = jnp.float32(-jnp.inf)
    top0 = jnp.max(logits, axis=-1, keepdims=True)
    thresh = top0
    for _ in range(_K - 1):
        thresh = jnp.max(jnp.where(logits < thresh, logits, neg),
                         axis=-1, keepdims=True)
    exps = jnp.where(logits >= thresh, jnp.exp(logits - top0),
                     jnp.float32(0.0))
    denom = jnp.sum(exps, axis=-1, keepdims=True)
    w_out_ref[...] = exps / denom


@jax.jit
def kernel(r_pooled, step_frac, hidden_norm, confidence, W1, b1, W2, b2):
    bn, d = r_pooled.shape
    h = W1.shape[1]
    e = W2.shape[1]
    w1_main = W1[:d]
    wsf = W1[d][None, :]
    whn = W1[d + 1][None, :]
    wcf = W1[d + 2][None, :]
    b1r = b1[None, :]
    sfv = jnp.asarray(step_frac, jnp.float32).reshape(1, 1)
    hc = jnp.stack([hidden_norm, confidence], axis=-1).astype(jnp.bfloat16)
    b2r = b2[None, :]

    grid = (bn // _BLK,)
    full = lambda *shape: pl.BlockSpec(shape, lambda i: (0,) * len(shape))
    rows = lambda w: pl.BlockSpec((_BLK, w), lambda i: (i, 0))
    out_shapes = (
        jax.ShapeDtypeStruct((bn, e), jnp.float32),
        jax.ShapeDtypeStruct((bn, e), jnp.float32),
    )
    weights, logits = pl.pallas_call(
        _router_block,
        grid=grid,
        in_specs=[
            rows(d),
            rows(2),
            full(1, 1),
            full(d, h),
            full(1, h),
            full(1, h),
            full(1, h),
            full(1, h),
            full(h, e),
            full(1, e),
        ],
        out_specs=(rows(e), rows(e)),
        out_shape=out_shapes,
    )(r_pooled, hc, sfv, w1_main, whn, wcf, wsf, b1r, W2, b2r)
    return weights, logits


# speedup vs baseline: 2.4549x; 1.1525x over previous
import functools

import jax
import jax.numpy as jnp
from jax.experimental import pallas as pl
from jax.experimental.pallas import tpu as pltpu

_K = 8
_BLK = 4096


def _router_block(r_ref, hc_ref, sf_ref, w1_ref, whn_ref, wcf_ref,
                  wsf_ref, b1_ref, w2_ref, b2_ref, w_out_ref, l_out_ref):
    bf = lambda x: x.astype(jnp.bfloat16)
    bfc = lambda x: x.astype(jnp.bfloat16).astype(jnp.float32)
    r = bf(r_ref[...])
    h = jnp.dot(r, bf(w1_ref[...]), preferred_element_type=jnp.float32)
    hc = hc_ref[...].astype(jnp.float32)
    h = h + hc[:, 0:1] * bfc(whn_ref[...])
    h = h + hc[:, 1:2] * bfc(wcf_ref[...])
    h = h + (bfc(sf_ref[...]) * bfc(wsf_ref[...]) + b1_ref[...])
    h = h * jax.nn.sigmoid(h)
    logits = jnp.dot(bf(h), bf(w2_ref[...]), preferred_element_type=jnp.float32)
    logits = logits + b2_ref[...]
    l_out_ref[...] = logits

    lt = jnp.transpose(logits)
    neg = jnp.float32(-jnp.inf)
    top0 = jnp.max(lt, axis=0, keepdims=True)
    thresh = top0
    for _ in range(_K - 1):
        thresh = jnp.max(jnp.where(lt < thresh, lt, neg),
                         axis=0, keepdims=True)
    exps = jnp.where(lt >= thresh, jnp.exp(lt - top0), jnp.float32(0.0))
    denom = jnp.sum(exps, axis=0, keepdims=True)
    w_out_ref[...] = jnp.transpose(exps / denom)


@jax.jit
def kernel(r_pooled, step_frac, hidden_norm, confidence, W1, b1, W2, b2):
    bn, d = r_pooled.shape
    h = W1.shape[1]
    e = W2.shape[1]
    w1_main = W1[:d]
    wsf = W1[d][None, :]
    whn = W1[d + 1][None, :]
    wcf = W1[d + 2][None, :]
    b1r = b1[None, :]
    sfv = jnp.asarray(step_frac, jnp.float32).reshape(1, 1)
    hc = jnp.stack([hidden_norm, confidence], axis=-1).astype(jnp.bfloat16)
    b2r = b2[None, :]

    grid = (bn // _BLK,)
    full = lambda *shape: pl.BlockSpec(shape, lambda i: (0,) * len(shape))
    rows = lambda w: pl.BlockSpec((_BLK, w), lambda i: (i, 0))
    out_shapes = (
        jax.ShapeDtypeStruct((bn, e), jnp.float32),
        jax.ShapeDtypeStruct((bn, e), jnp.float32),
    )
    weights, logits = pl.pallas_call(
        _router_block,
        grid=grid,
        in_specs=[
            rows(d),
            rows(2),
            full(1, 1),
            full(d, h),
            full(1, h),
            full(1, h),
            full(1, h),
            full(1, h),
            full(h, e),
            full(1, e),
        ],
        out_specs=(rows(e), rows(e)),
        out_shape=out_shapes,
    )(r_pooled, hc, sfv, w1_main, whn, wcf, wsf, b1r, W2, b2r)
    return weights, logits
